# baseline (speedup 1.0000x reference)
import functools

import jax
import jax.numpy as jnp
from jax import lax
from jax.experimental import pallas as pl
from jax.experimental.pallas import tpu as pltpu
from jax.experimental.pallas import tpu_sc as plsc

N = 10000
NP = 10112
E = 320000
EL = E + N
HID = 128
NG = 16
NW = 32
K = 128
CH0 = 71
CH1 = 91
EPAD = NW // 2 * K * (CH0 + CH1)
ROWS_PT = NP // 16


def _leaky(v):
    return jnp.where(v > 0.0, v, 0.2 * v)


def _lin_attn(h, as_ref, ad_ref, als_ref, ald_ref, c_ref):
    als = jnp.dot(h, as_ref[...], preferred_element_type=jnp.float32)
    ald = jnp.dot(h, ad_ref[...], preferred_element_type=jnp.float32)
    als_ref[...] = als
    ald_ref[...] = ald
    c_ref[...] = _leaky(jnp.max(als) + ald)


def _k1_body(x_ref, w_ref, as_ref, ad_ref, h_ref, als_ref, ald_ref, c_ref):
    h = jnp.dot(x_ref[...], w_ref[...], preferred_element_type=jnp.float32)
    h_ref[...] = h
    _lin_attn(h, as_ref, ad_ref, als_ref, ald_ref, c_ref)


def _combine(p_ref, d0_ref, d1_ref, b_ref):
    den = d0_ref[...] + d1_ref[...] + 1e-16
    num = p_ref[0] + p_ref[1]
    u = num / den + b_ref[...][None, :]
    return jnp.where(u > 0.0, u, jnp.exp(jnp.minimum(u, 0.0)) - 1.0)


def _k2_body(p_ref, d0_ref, d1_ref, b_ref, w_ref, as_ref, ad_ref,
             h_ref, als_ref, ald_ref, c_ref):
    u = _combine(p_ref, d0_ref, d1_ref, b_ref)
    h = jnp.dot(u, w_ref[...], preferred_element_type=jnp.float32)
    h_ref[...] = h
    _lin_attn(h, as_ref, ad_ref, als_ref, ald_ref, c_ref)


def _k3_body(p_ref, d0_ref, d1_ref, b_ref, batch_ref, out_ref):
    u = _combine(p_ref, d0_ref, d1_ref, b_ref)
    bt = batch_ref[...]
    oh = (lax.broadcasted_iota(jnp.int32, (NG, NP), 0) == bt).astype(jnp.float32)
    sums = jnp.dot(oh, u, preferred_element_type=jnp.float32)
    counts = jnp.sum(oh, axis=1, keepdims=True)
    out_ref[...] = sums / jnp.maximum(counts, 1.0)


def _tc_prep1(xp, W, a_s, a_d):
    return pl.pallas_call(
        _k1_body,
        out_shape=(
            jax.ShapeDtypeStruct((NP, HID), jnp.float32),
            jax.ShapeDtypeStruct((NP, 1), jnp.float32),
            jax.ShapeDtypeStruct((NP, 1), jnp.float32),
            jax.ShapeDtypeStruct((NP, 1), jnp.float32),
        ),
    )(xp, W, a_s, a_d)


def _tc_prep2(p, d0, d1, b, W, a_s, a_d):
    return pl.pallas_call(
        _k2_body,
        out_shape=(
            jax.ShapeDtypeStruct((NP, HID), jnp.float32),
            jax.ShapeDtypeStruct((NP, 1), jnp.float32),
            jax.ShapeDtypeStruct((NP, 1), jnp.float32),
            jax.ShapeDtypeStruct((NP, 1), jnp.float32),
        ),
    )(p, d0, d1, b, W, a_s, a_d)


def _tc_pool(p, d0, d1, b, batch2):
    return pl.pallas_call(
        _k3_body,
        out_shape=jax.ShapeDtypeStruct((NG, HID), jnp.float32),
    )(p, d0, d1, b, batch2)


def _compute_ex(src_v, dst_v, ex_v, als_v, ald_v, c_v):
    def grp(i, _):
        si = src_v[pl.ds(i * 16, 16)]
        di = dst_v[pl.ds(i * 16, 16)]
        e = plsc.load_gather(als_v, [si]) + plsc.load_gather(ald_v, [di])
        ex = jnp.exp(_leaky(e) - plsc.load_gather(c_v, [di]))
        ex_v[pl.ds(i * 16, 16)] = ex
        return 0
    lax.fori_loop(0, K // 16, grp, 0)


def _scale_rows_half(rows_v, ex_v, h):
    def mul(i, _):
        exv = ex_v[pl.ds(h * (K // 2) + i * 16, 16)]
        for l in range(16):
            kk = i * 16 + l
            s = exv[l]
            for j in range(8):
                rows_v[kk, pl.ds(j * 16, 16)] = (
                    rows_v[kk, pl.ds(j * 16, 16)] * s)
        return 0
    lax.fori_loop(0, K // 32, mul, 0)


def _fill_idx4(idx4, src_stage, dst_stage):
    for hh in range(2):
        for qq in range(K // 2 // 16):
            idx4[hh, pl.ds(qq * 16, 16)] = (
                src_stage[pl.ds(hh * (K // 2) + qq * 16, 16)])
            idx4[2 + hh, pl.ds(qq * 16, 16)] = (
                dst_stage[pl.ds(hh * (K // 2) + qq * 16, 16)])


def _sc_edge_pass(h, als, ald, ctab, srcp, dstp):
    mesh = plsc.VectorSubcoreMesh(core_axis_name="c", subcore_axis_name="s")

    @functools.partial(
        pl.kernel,
        out_type=(
            jax.ShapeDtypeStruct((2, NP, HID), jnp.float32),
            jax.ShapeDtypeStruct((NP,), jnp.float32),
            jax.ShapeDtypeStruct((NP,), jnp.float32),
        ),
        mesh=mesh,
        compiler_params=pltpu.CompilerParams(needs_layout_passes=False),
        scratch_types=[
            pltpu.VMEM((NP,), jnp.float32),
            pltpu.VMEM((NP,), jnp.float32),
            pltpu.VMEM((NP,), jnp.float32),
            pltpu.VMEM((K,), jnp.int32),
            pltpu.VMEM((K,), jnp.int32),
            pltpu.VMEM((K,), jnp.int32),
            pltpu.VMEM((K,), jnp.int32),
            pltpu.VMEM((K,), jnp.float32),
            pltpu.VMEM((K,), jnp.float32),
            pltpu.VMEM((K // 2, HID), jnp.float32),
            pltpu.VMEM((K // 2, HID), jnp.float32),
            pltpu.VMEM((4, K // 2), jnp.int32),
            pltpu.VMEM((4, K // 2), jnp.int32),
            pltpu.VMEM_SHARED((NP, HID), jnp.float32),
            pltpu.VMEM_SHARED((NP,), jnp.float32),
            pltpu.SemaphoreType.DMA,
            pltpu.SemaphoreType.DMA,
            pltpu.SemaphoreType.DMA,
            pltpu.SemaphoreType.DMA,
            pltpu.SemaphoreType.DMA,
            pltpu.SemaphoreType.DMA,
            pltpu.SemaphoreType.DMA,
        ],
    )
    def k(h_hbm, als_hbm, ald_hbm, c_hbm, src_hbm, dst_hbm,
          p_out, d0_out, d1_out,
          als_v, ald_v, c_v, src0, dst0, src1, dst1, ex0, ex1,
          rowsa, rowsb, idx4a, idx4b,
          acc, den, semga, semgb, semsca, semscb, semi0, semi1, semd):
        cid = lax.axis_index("c")
        sid = lax.axis_index("s")
        wid = cid * 16 + sid
        srcs = (src0, src1)
        dsts = (dst0, dst1)
        exs = (ex0, ex1)
        idx4s = (idx4a, idx4b)
        semi = (semi0, semi1)

        pltpu.sync_copy(als_hbm, als_v)
        pltpu.sync_copy(ald_hbm, ald_v)
        pltpu.sync_copy(c_hbm, c_v)

        zero16 = jnp.zeros((16,), jnp.float32)

        def zrow(i, _):
            for j in range(8):
                rowsa[i, pl.ds(j * 16, 16)] = zero16
            return 0
        lax.fori_loop(0, K // 2, zrow, 0)

        row0 = sid * ROWS_PT
        for q in range(ROWS_PT // (K // 2)):
            pltpu.sync_copy(rowsa, acc.at[pl.ds(row0 + q * (K // 2), K // 2)])
        _rem = ROWS_PT % (K // 2)
        if _rem:
            pltpu.sync_copy(
                rowsa.at[pl.ds(0, _rem)],
                acc.at[pl.ds(row0 + (ROWS_PT // (K // 2)) * (K // 2), _rem)])
        for t in range(NP // 128 // 16 + 1):
            cchunk = sid + 16 * t

            @pl.when(cchunk < NP // 128)
            def _():
                pltpu.sync_copy(rowsa.at[0], den.at[pl.ds(cchunk * 128, 128)])
        plsc.subcore_barrier()

        def run_pipeline(ebase, ch):
            def step(g, cur, nxt, has_next, next2_pred):
                if has_next:
                    pltpu.make_async_copy(
                        src_hbm.at[pl.ds(0, K)], srcs[nxt], semi[nxt]).wait()
                    pltpu.make_async_copy(
                        dst_hbm.at[pl.ds(0, K)], dsts[nxt], semi[nxt]).wait()
                    _fill_idx4(idx4s[nxt], srcs[nxt], dsts[nxt])
                    _compute_ex(srcs[nxt], dsts[nxt], exs[nxt],
                                als_v, ald_v, c_v)
                pltpu.make_async_copy(
                    h_hbm.at[idx4s[cur].at[0]], rowsa, semga).wait()
                _scale_rows_half(rowsa, exs[cur], 0)
                da = pltpu.async_copy(rowsa, acc.at[idx4s[cur].at[2]],
                                      semsca, add=True)
                pltpu.make_async_copy(
                    h_hbm.at[idx4s[cur].at[1]], rowsb, semgb).wait()
                _scale_rows_half(rowsb, exs[cur], 1)
                db = pltpu.async_copy(rowsb, acc.at[idx4s[cur].at[3]],
                                      semscb, add=True)
                dd = pltpu.async_copy(exs[cur], den.at[dsts[cur]], semd,
                                      add=True)
                da.wait()
                if has_next:
                    pltpu.async_copy(h_hbm.at[idx4s[nxt].at[0]], rowsa, semga)
                db.wait()
                if has_next:
                    pltpu.async_copy(h_hbm.at[idx4s[nxt].at[1]], rowsb, semgb)
                dd.wait()
                if next2_pred is not None:
                    @pl.when(next2_pred)
                    def _():
                        off = ebase + (g + 2) * K
                        pltpu.async_copy(src_hbm.at[pl.ds(off, K)],
                                         srcs[cur], semi[cur])
                        pltpu.async_copy(dst_hbm.at[pl.ds(off, K)],
                                         dsts[cur], semi[cur])

            pltpu.sync_copy(src_hbm.at[pl.ds(ebase, K)], src0)
            pltpu.sync_copy(dst_hbm.at[pl.ds(ebase, K)], dst0)
            _fill_idx4(idx4a, src0, dst0)
            pltpu.async_copy(h_hbm.at[idx4a.at[0]], rowsa, semga)
            pltpu.async_copy(h_hbm.at[idx4a.at[1]], rowsb, semgb)
            pltpu.async_copy(src_hbm.at[pl.ds(ebase + K, K)], src1, semi1)
            pltpu.async_copy(dst_hbm.at[pl.ds(ebase + K, K)], dst1, semi1)
            _compute_ex(src0, dst0, ex0, als_v, ald_v, c_v)

            def pair(tt, _):
                g0 = 2 * tt
                step(g0, 0, 1, True, g0 + 2 < ch)
                step(g0 + 1, 1, 0, True, g0 + 3 < ch)
                return 0
            lax.fori_loop(0, ch // 2, pair, 0)
            step(ch - 1, 0, 1, False, None)

        @pl.when(cid == 0)
        def _():
            run_pipeline(sid * (K * CH0), CH0)

        @pl.when(cid == 1)
        def _():
            run_pipeline(16 * K * CH0 + sid * (K * CH1), CH1)

        plsc.subcore_barrier()
        pltpu.sync_copy(acc.at[pl.ds(row0, ROWS_PT)],
                        p_out.at[cid, pl.ds(row0, ROWS_PT)])
        for t in range(NP // K // 16 + 1):
            cchunk = sid + 16 * t

            @pl.when(jnp.logical_and(cchunk < NP // K, cid == 0))
            def _():
                pltpu.sync_copy(den.at[pl.ds(cchunk * K, K)],
                                d0_out.at[pl.ds(cchunk * K, K)])

            @pl.when(jnp.logical_and(cchunk < NP // K, cid == 1))
            def _():
                pltpu.sync_copy(den.at[pl.ds(cchunk * K, K)],
                                d1_out.at[pl.ds(cchunk * K, K)])

    return k(h, als, ald, ctab, srcp, dstp)


def kernel(x, edge_index, batch, W1, a_src1, a_dst1, b1, W2, a_src2, a_dst2, b2):
    loop = jnp.arange(N, dtype=jnp.int32)
    pad = jnp.full((EPAD - EL,), N, jnp.int32)
    srcp = jnp.concatenate([edge_index[0].astype(jnp.int32), loop, pad])
    dstp = jnp.concatenate([edge_index[1].astype(jnp.int32), loop, pad])
    xp = jnp.concatenate([x, jnp.zeros((NP - N, x.shape[1]), jnp.float32)], axis=0)
    batp = jnp.concatenate(
        [batch.astype(jnp.int32), jnp.full((NP - N,), NG, jnp.int32)]
    ).reshape(1, NP)

    h1, als1, ald1, c1 = _tc_prep1(xp, W1, a_src1[:, None], a_dst1[:, None])
    p1, d1a, d1b = _sc_edge_pass(
        h1, als1.reshape(NP), ald1.reshape(NP), c1.reshape(NP), srcp, dstp)
    h2, als2, ald2, c2 = _tc_prep2(p1, d1a[:, None], d1b[:, None], b1,
                                   W2, a_src2[:, None], a_dst2[:, None])
    p2, d2a, d2b = _sc_edge_pass(
        h2, als2.reshape(NP), ald2.reshape(NP), c2.reshape(NP), srcp, dstp)
    return _tc_pool(p2, d2a[:, None], d2b[:, None], b2, batp)


# speedup vs baseline: 1.0948x; 1.0948x over previous
import functools

import jax
import jax.numpy as jnp
from jax import lax
from jax.experimental import pallas as pl
from jax.experimental.pallas import tpu as pltpu
from jax.experimental.pallas import tpu_sc as plsc

N = 10000
NP = 10112
E = 320000
EL = E + N
HID = 128
NG = 16
NW = 32
K = 128
CH0 = 91
CH1 = 71
EPAD = NW // 2 * K * (CH0 + CH1)
ROWS_PT = NP // 16


def _leaky(v):
    return jnp.where(v > 0.0, v, 0.2 * v)


def _lin_attn(h, as_ref, ad_ref, als_ref, ald_ref, c_ref):
    als = jnp.dot(h, as_ref[...], preferred_element_type=jnp.float32)
    ald = jnp.dot(h, ad_ref[...], preferred_element_type=jnp.float32)
    als_ref[...] = als
    ald_ref[...] = ald
    c_ref[...] = _leaky(jnp.max(als) + ald)


def _k1_body(x_ref, w_ref, as_ref, ad_ref, h_ref, als_ref, ald_ref, c_ref):
    h = jnp.dot(x_ref[...], w_ref[...], preferred_element_type=jnp.float32)
    h_ref[...] = h
    _lin_attn(h, as_ref, ad_ref, als_ref, ald_ref, c_ref)


def _combine(p_ref, d0_ref, d1_ref, b_ref):
    den = d0_ref[...] + d1_ref[...] + 1e-16
    num = p_ref[0] + p_ref[1]
    u = num / den + b_ref[...][None, :]
    return jnp.where(u > 0.0, u, jnp.exp(jnp.minimum(u, 0.0)) - 1.0)


def _k2_body(p_ref, d0_ref, d1_ref, b_ref, w_ref, as_ref, ad_ref,
             h_ref, als_ref, ald_ref, c_ref):
    u = _combine(p_ref, d0_ref, d1_ref, b_ref)
    h = jnp.dot(u, w_ref[...], preferred_element_type=jnp.float32)
    h_ref[...] = h
    _lin_attn(h, as_ref, ad_ref, als_ref, ald_ref, c_ref)


def _k3_body(p_ref, d0_ref, d1_ref, b_ref, batch_ref, out_ref):
    u = _combine(p_ref, d0_ref, d1_ref, b_ref)
    bt = batch_ref[...]
    oh = (lax.broadcasted_iota(jnp.int32, (NG, NP), 0) == bt).astype(jnp.float32)
    sums = jnp.dot(oh, u, preferred_element_type=jnp.float32)
    counts = jnp.sum(oh, axis=1, keepdims=True)
    out_ref[...] = sums / jnp.maximum(counts, 1.0)


def _tc_prep1(xp, W, a_s, a_d):
    return pl.pallas_call(
        _k1_body,
        out_shape=(
            jax.ShapeDtypeStruct((NP, HID), jnp.float32),
            jax.ShapeDtypeStruct((NP, 1), jnp.float32),
            jax.ShapeDtypeStruct((NP, 1), jnp.float32),
            jax.ShapeDtypeStruct((NP, 1), jnp.float32),
        ),
    )(xp, W, a_s, a_d)


def _tc_prep2(p, d0, d1, b, W, a_s, a_d):
    return pl.pallas_call(
        _k2_body,
        out_shape=(
            jax.ShapeDtypeStruct((NP, HID), jnp.float32),
            jax.ShapeDtypeStruct((NP, 1), jnp.float32),
            jax.ShapeDtypeStruct((NP, 1), jnp.float32),
            jax.ShapeDtypeStruct((NP, 1), jnp.float32),
        ),
    )(p, d0, d1, b, W, a_s, a_d)


def _tc_pool(p, d0, d1, b, batch2):
    return pl.pallas_call(
        _k3_body,
        out_shape=jax.ShapeDtypeStruct((NG, HID), jnp.float32),
    )(p, d0, d1, b, batch2)


def _compute_ex(src_v, dst_v, ex_v, als_v, ald_v, c_v):
    def grp(i, _):
        si = src_v[pl.ds(i * 16, 16)]
        di = dst_v[pl.ds(i * 16, 16)]
        e = plsc.load_gather(als_v, [si]) + plsc.load_gather(ald_v, [di])
        ex = jnp.exp(_leaky(e) - plsc.load_gather(c_v, [di]))
        ex_v[pl.ds(i * 16, 16)] = ex
        return 0
    lax.fori_loop(0, K // 16, grp, 0)


def _scale_rows_half(rows_v, ex_v, h):
    def mul(i, _):
        exv = ex_v[pl.ds(h * (K // 2) + i * 16, 16)]
        for l in range(16):
            kk = i * 16 + l
            s = exv[l]
            for j in range(8):
                rows_v[kk, pl.ds(j * 16, 16)] = (
                    rows_v[kk, pl.ds(j * 16, 16)] * s)
        return 0
    lax.fori_loop(0, K // 32, mul, 0)


def _fill_idx4(idx4, src_stage, dst_stage):
    for hh in range(2):
        for qq in range(K // 2 // 16):
            idx4[hh, pl.ds(qq * 16, 16)] = (
                src_stage[pl.ds(hh * (K // 2) + qq * 16, 16)])
            idx4[2 + hh, pl.ds(qq * 16, 16)] = (
                dst_stage[pl.ds(hh * (K // 2) + qq * 16, 16)])


def _sc_edge_pass(h, als, ald, ctab, srcp, dstp):
    mesh = plsc.VectorSubcoreMesh(core_axis_name="c", subcore_axis_name="s")

    @functools.partial(
        pl.kernel,
        out_type=(
            jax.ShapeDtypeStruct((2, NP, HID), jnp.float32),
            jax.ShapeDtypeStruct((NP,), jnp.float32),
            jax.ShapeDtypeStruct((NP,), jnp.float32),
        ),
        mesh=mesh,
        compiler_params=pltpu.CompilerParams(needs_layout_passes=False),
        scratch_types=[
            pltpu.VMEM((NP,), jnp.float32),
            pltpu.VMEM((NP,), jnp.float32),
            pltpu.VMEM((NP,), jnp.float32),
            pltpu.VMEM((K,), jnp.int32),
            pltpu.VMEM((K,), jnp.int32),
            pltpu.VMEM((K,), jnp.int32),
            pltpu.VMEM((K,), jnp.int32),
            pltpu.VMEM((K,), jnp.float32),
            pltpu.VMEM((K,), jnp.float32),
            pltpu.VMEM((K // 2, HID), jnp.float32),
            pltpu.VMEM((K // 2, HID), jnp.float32),
            pltpu.VMEM((4, K // 2), jnp.int32),
            pltpu.VMEM((4, K // 2), jnp.int32),
            pltpu.VMEM_SHARED((NP, HID), jnp.float32),
            pltpu.VMEM_SHARED((NP,), jnp.float32),
            pltpu.SemaphoreType.DMA,
            pltpu.SemaphoreType.DMA,
            pltpu.SemaphoreType.DMA,
            pltpu.SemaphoreType.DMA,
            pltpu.SemaphoreType.DMA,
            pltpu.SemaphoreType.DMA,
            pltpu.SemaphoreType.DMA,
        ],
    )
    def k(h_hbm, als_hbm, ald_hbm, c_hbm, src_hbm, dst_hbm,
          p_out, d0_out, d1_out,
          als_v, ald_v, c_v, src0, dst0, src1, dst1, ex0, ex1,
          rowsa, rowsb, idx4a, idx4b,
          acc, den, semga, semgb, semsca, semscb, semi0, semi1, semd):
        cid = lax.axis_index("c")
        sid = lax.axis_index("s")
        wid = cid * 16 + sid
        srcs = (src0, src1)
        dsts = (dst0, dst1)
        exs = (ex0, ex1)
        idx4s = (idx4a, idx4b)
        semi = (semi0, semi1)

        pltpu.sync_copy(als_hbm, als_v)
        pltpu.sync_copy(ald_hbm, ald_v)
        pltpu.sync_copy(c_hbm, c_v)

        zero16 = jnp.zeros((16,), jnp.float32)

        def zrow(i, _):
            for j in range(8):
                rowsa[i, pl.ds(j * 16, 16)] = zero16
            return 0
        lax.fori_loop(0, K // 2, zrow, 0)

        row0 = sid * ROWS_PT
        for q in range(ROWS_PT // (K // 2)):
            pltpu.sync_copy(rowsa, acc.at[pl.ds(row0 + q * (K // 2), K // 2)])
        _rem = ROWS_PT % (K // 2)
        if _rem:
            pltpu.sync_copy(
                rowsa.at[pl.ds(0, _rem)],
                acc.at[pl.ds(row0 + (ROWS_PT // (K // 2)) * (K // 2), _rem)])
        for t in range(NP // 128 // 16 + 1):
            cchunk = sid + 16 * t

            @pl.when(cchunk < NP // 128)
            def _():
                pltpu.sync_copy(rowsa.at[0], den.at[pl.ds(cchunk * 128, 128)])
        plsc.subcore_barrier()

        def run_pipeline(ebase, ch):
            def step(g, cur, nxt, has_next, next2_pred):
                if has_next:
                    pltpu.make_async_copy(
                        src_hbm.at[pl.ds(0, K)], srcs[nxt], semi[nxt]).wait()
                    pltpu.make_async_copy(
                        dst_hbm.at[pl.ds(0, K)], dsts[nxt], semi[nxt]).wait()
                    _fill_idx4(idx4s[nxt], srcs[nxt], dsts[nxt])
                    _compute_ex(srcs[nxt], dsts[nxt], exs[nxt],
                                als_v, ald_v, c_v)
                pltpu.make_async_copy(
                    h_hbm.at[idx4s[cur].at[0]], rowsa, semga).wait()
                _scale_rows_half(rowsa, exs[cur], 0)
                da = pltpu.async_copy(rowsa, acc.at[idx4s[cur].at[2]],
                                      semsca, add=True)
                pltpu.make_async_copy(
                    h_hbm.at[idx4s[cur].at[1]], rowsb, semgb).wait()
                _scale_rows_half(rowsb, exs[cur], 1)
                db = pltpu.async_copy(rowsb, acc.at[idx4s[cur].at[3]],
                                      semscb, add=True)
                dd = pltpu.async_copy(exs[cur], den.at[dsts[cur]], semd,
                                      add=True)
                da.wait()
                if has_next:
                    pltpu.async_copy(h_hbm.at[idx4s[nxt].at[0]], rowsa, semga)
                db.wait()
                if has_next:
                    pltpu.async_copy(h_hbm.at[idx4s[nxt].at[1]], rowsb, semgb)
                dd.wait()
                if next2_pred is not None:
                    @pl.when(next2_pred)
                    def _():
                        off = ebase + (g + 2) * K
                        pltpu.async_copy(src_hbm.at[pl.ds(off, K)],
                                         srcs[cur], semi[cur])
                        pltpu.async_copy(dst_hbm.at[pl.ds(off, K)],
                                         dsts[cur], semi[cur])

            pltpu.sync_copy(src_hbm.at[pl.ds(ebase, K)], src0)
            pltpu.sync_copy(dst_hbm.at[pl.ds(ebase, K)], dst0)
            _fill_idx4(idx4a, src0, dst0)
            pltpu.async_copy(h_hbm.at[idx4a.at[0]], rowsa, semga)
            pltpu.async_copy(h_hbm.at[idx4a.at[1]], rowsb, semgb)
            pltpu.async_copy(src_hbm.at[pl.ds(ebase + K, K)], src1, semi1)
            pltpu.async_copy(dst_hbm.at[pl.ds(ebase + K, K)], dst1, semi1)
            _compute_ex(src0, dst0, ex0, als_v, ald_v, c_v)

            def pair(tt, _):
                g0 = 2 * tt
                step(g0, 0, 1, True, g0 + 2 < ch)
                step(g0 + 1, 1, 0, True, g0 + 3 < ch)
                return 0
            lax.fori_loop(0, ch // 2, pair, 0)
            step(ch - 1, 0, 1, False, None)

        @pl.when(cid == 0)
        def _():
            run_pipeline(sid * (K * CH0), CH0)

        @pl.when(cid == 1)
        def _():
            run_pipeline(16 * K * CH0 + sid * (K * CH1), CH1)

        plsc.subcore_barrier()
        pltpu.sync_copy(acc.at[pl.ds(row0, ROWS_PT)],
                        p_out.at[cid, pl.ds(row0, ROWS_PT)])
        for t in range(NP // K // 16 + 1):
            cchunk = sid + 16 * t

            @pl.when(jnp.logical_and(cchunk < NP // K, cid == 0))
            def _():
                pltpu.sync_copy(den.at[pl.ds(cchunk * K, K)],
                                d0_out.at[pl.ds(cchunk * K, K)])

            @pl.when(jnp.logical_and(cchunk < NP // K, cid == 1))
            def _():
                pltpu.sync_copy(den.at[pl.ds(cchunk * K, K)],
                                d1_out.at[pl.ds(cchunk * K, K)])

    return k(h, als, ald, ctab, srcp, dstp)


def kernel(x, edge_index, batch, W1, a_src1, a_dst1, b1, W2, a_src2, a_dst2, b2):
    loop = jnp.arange(N, dtype=jnp.int32)
    pad = jnp.full((EPAD - EL,), N, jnp.int32)
    srcp = jnp.concatenate([edge_index[0].astype(jnp.int32), loop, pad])
    dstp = jnp.concatenate([edge_index[1].astype(jnp.int32), loop, pad])
    xp = jnp.concatenate([x, jnp.zeros((NP - N, x.shape[1]), jnp.float32)], axis=0)
    batp = jnp.concatenate(
        [batch.astype(jnp.int32), jnp.full((NP - N,), NG, jnp.int32)]
    ).reshape(1, NP)

    h1, als1, ald1, c1 = _tc_prep1(xp, W1, a_src1[:, None], a_dst1[:, None])
    p1, d1a, d1b = _sc_edge_pass(
        h1, als1.reshape(NP), ald1.reshape(NP), c1.reshape(NP), srcp, dstp)
    h2, als2, ald2, c2 = _tc_prep2(p1, d1a[:, None], d1b[:, None], b1,
                                   W2, a_src2[:, None], a_dst2[:, None])
    p2, d2a, d2b = _sc_edge_pass(
        h2, als2.reshape(NP), ald2.reshape(NP), c2.reshape(NP), srcp, dstp)
    return _tc_pool(p2, d2a[:, None], d2b[:, None], b2, batp)


# speedup vs baseline: 1.0952x; 1.0004x over previous
import functools

import jax
import jax.numpy as jnp
from jax import lax
from jax.experimental import pallas as pl
from jax.experimental.pallas import tpu as pltpu
from jax.experimental.pallas import tpu_sc as plsc

N = 10000
NP = 10112
E = 320000
EL = E + N
HID = 128
NG = 16
NW = 32
K = 128
CH0 = 91
CH1 = 71
EPAD = NW // 2 * K * (CH0 + CH1)
ROWS_PT = NP // 16


def _leaky(v):
    return jnp.where(v > 0.0, v, 0.2 * v)


def _lin_attn(h, as_ref, ad_ref, als_ref, ald_ref, c_ref):
    als = jnp.dot(h, as_ref[...], preferred_element_type=jnp.float32)
    ald = jnp.dot(h, ad_ref[...], preferred_element_type=jnp.float32)
    als_ref[...] = als
    ald_ref[...] = ald
    c_ref[...] = _leaky(jnp.max(als) + ald)


def _k1_body(x_ref, w_ref, as_ref, ad_ref, h_ref, als_ref, ald_ref, c_ref):
    h = jnp.dot(x_ref[...], w_ref[...], preferred_element_type=jnp.float32)
    h_ref[...] = h
    _lin_attn(h, as_ref, ad_ref, als_ref, ald_ref, c_ref)


def _combine(p_ref, d0_ref, d1_ref, b_ref):
    den = d0_ref[...] + d1_ref[...] + 1e-16
    num = p_ref[0] + p_ref[1]
    u = num / den + b_ref[...][None, :]
    return jnp.where(u > 0.0, u, jnp.exp(jnp.minimum(u, 0.0)) - 1.0)


def _k2_body(p_ref, d0_ref, d1_ref, b_ref, w_ref, as_ref, ad_ref,
             h_ref, als_ref, ald_ref, c_ref):
    u = _combine(p_ref, d0_ref, d1_ref, b_ref)
    h = jnp.dot(u, w_ref[...], preferred_element_type=jnp.float32)
    h_ref[...] = h
    _lin_attn(h, as_ref, ad_ref, als_ref, ald_ref, c_ref)


def _k3_body(p_ref, d0_ref, d1_ref, b_ref, batch_ref, out_ref):
    u = _combine(p_ref, d0_ref, d1_ref, b_ref)
    bt = batch_ref[...]
    oh = (lax.broadcasted_iota(jnp.int32, (NG, NP), 0) == bt).astype(jnp.float32)
    sums = jnp.dot(oh, u, preferred_element_type=jnp.float32)
    counts = jnp.sum(oh, axis=1, keepdims=True)
    out_ref[...] = sums / jnp.maximum(counts, 1.0)


def _tc_prep1(xp, W, a_s, a_d):
    return pl.pallas_call(
        _k1_body,
        out_shape=(
            jax.ShapeDtypeStruct((NP, HID), jnp.float32),
            jax.ShapeDtypeStruct((NP, 1), jnp.float32),
            jax.ShapeDtypeStruct((NP, 1), jnp.float32),
            jax.ShapeDtypeStruct((NP, 1), jnp.float32),
        ),
    )(xp, W, a_s, a_d)


def _tc_prep2(p, d0, d1, b, W, a_s, a_d):
    return pl.pallas_call(
        _k2_body,
        out_shape=(
            jax.ShapeDtypeStruct((NP, HID), jnp.float32),
            jax.ShapeDtypeStruct((NP, 1), jnp.float32),
            jax.ShapeDtypeStruct((NP, 1), jnp.float32),
            jax.ShapeDtypeStruct((NP, 1), jnp.float32),
        ),
    )(p, d0, d1, b, W, a_s, a_d)


def _tc_pool(p, d0, d1, b, batch2):
    return pl.pallas_call(
        _k3_body,
        out_shape=jax.ShapeDtypeStruct((NG, HID), jnp.float32),
    )(p, d0, d1, b, batch2)


def _compute_ex(src_v, dst_v, ex_v, als_v, ald_v, c_v):
    def grp(i, _):
        si = src_v[pl.ds(i * 16, 16)]
        di = dst_v[pl.ds(i * 16, 16)]
        e = plsc.load_gather(als_v, [si]) + plsc.load_gather(ald_v, [di])
        ex = jnp.exp(_leaky(e) - plsc.load_gather(c_v, [di]))
        ex_v[pl.ds(i * 16, 16)] = ex
        return 0
    lax.fori_loop(0, K // 16, grp, 0)


def _scale_rows_half(rows_v, ex_v, h):
    def mul(i, _):
        exv = ex_v[pl.ds(h * (K // 2) + i * 16, 16)]
        for l in range(16):
            kk = i * 16 + l
            s = exv[l]
            for j in range(8):
                rows_v[kk, pl.ds(j * 16, 16)] = (
                    rows_v[kk, pl.ds(j * 16, 16)] * s)
        return 0
    lax.fori_loop(0, K // 32, mul, 0)


def _fill_idx4(idx4, src_stage, dst_stage):
    for hh in range(2):
        for qq in range(K // 2 // 16):
            idx4[hh, pl.ds(qq * 16, 16)] = (
                src_stage[pl.ds(hh * (K // 2) + qq * 16, 16)])
            idx4[2 + hh, pl.ds(qq * 16, 16)] = (
                dst_stage[pl.ds(hh * (K // 2) + qq * 16, 16)])


def _sc_edge_pass(h, als, ald, ctab, srcp, dstp):
    mesh = plsc.VectorSubcoreMesh(core_axis_name="c", subcore_axis_name="s")

    @functools.partial(
        pl.kernel,
        out_type=(
            jax.ShapeDtypeStruct((2, NP, HID), jnp.float32),
            jax.ShapeDtypeStruct((NP,), jnp.float32),
            jax.ShapeDtypeStruct((NP,), jnp.float32),
        ),
        mesh=mesh,
        compiler_params=pltpu.CompilerParams(needs_layout_passes=False),
        scratch_types=[
            pltpu.VMEM((NP,), jnp.float32),
            pltpu.VMEM((NP,), jnp.float32),
            pltpu.VMEM((NP,), jnp.float32),
            pltpu.VMEM((K,), jnp.int32),
            pltpu.VMEM((K,), jnp.int32),
            pltpu.VMEM((K,), jnp.int32),
            pltpu.VMEM((K,), jnp.int32),
            pltpu.VMEM((K,), jnp.float32),
            pltpu.VMEM((K,), jnp.float32),
            pltpu.VMEM((K // 2, HID), jnp.float32),
            pltpu.VMEM((K // 2, HID), jnp.float32),
            pltpu.VMEM((4, K // 2), jnp.int32),
            pltpu.VMEM((4, K // 2), jnp.int32),
            pltpu.VMEM_SHARED((NP, HID), jnp.float32),
            pltpu.VMEM_SHARED((NP,), jnp.float32),
            pltpu.SemaphoreType.DMA,
            pltpu.SemaphoreType.DMA,
            pltpu.SemaphoreType.DMA,
            pltpu.SemaphoreType.DMA,
            pltpu.SemaphoreType.DMA,
            pltpu.SemaphoreType.DMA,
            pltpu.SemaphoreType.DMA,
        ],
    )
    def k(h_hbm, als_hbm, ald_hbm, c_hbm, src_hbm, dst_hbm,
          p_out, d0_out, d1_out,
          als_v, ald_v, c_v, src0, dst0, src1, dst1, ex0, ex1,
          rowsa, rowsb, idx4a, idx4b,
          acc, den, semga, semgb, semsca, semscb, semi0, semi1, semd):
        cid = lax.axis_index("c")
        sid = lax.axis_index("s")
        srcs = (src0, src1)
        dsts = (dst0, dst1)
        exs = (ex0, ex1)
        idx4s = (idx4a, idx4b)
        semi = (semi0, semi1)

        pltpu.sync_copy(als_hbm, als_v)
        pltpu.sync_copy(ald_hbm, ald_v)
        pltpu.sync_copy(c_hbm, c_v)

        zero16 = jnp.zeros((16,), jnp.float32)

        def zrow(i, _):
            for j in range(8):
                rowsa[i, pl.ds(j * 16, 16)] = zero16
            return 0
        lax.fori_loop(0, K // 2, zrow, 0)

        row0 = sid * ROWS_PT
        for q in range(ROWS_PT // (K // 2)):
            pltpu.sync_copy(rowsa, acc.at[pl.ds(row0 + q * (K // 2), K // 2)])
        _rem = ROWS_PT % (K // 2)
        if _rem:
            pltpu.sync_copy(
                rowsa.at[pl.ds(0, _rem)],
                acc.at[pl.ds(row0 + (ROWS_PT // (K // 2)) * (K // 2), _rem)])
        for t in range(NP // 128 // 16 + 1):
            cchunk = sid + 16 * t

            @pl.when(cchunk < NP // 128)
            def _():
                pltpu.sync_copy(rowsa.at[0], den.at[pl.ds(cchunk * 128, 128)])
        plsc.subcore_barrier()

        def run_pipeline(ebase, ch):
            def step(g, cur, nxt, has_next, next2_pred):
                if has_next:
                    pltpu.make_async_copy(
                        src_hbm.at[pl.ds(0, K)], srcs[nxt], semi[nxt]).wait()
                    pltpu.make_async_copy(
                        dst_hbm.at[pl.ds(0, K)], dsts[nxt], semi[nxt]).wait()
                    _fill_idx4(idx4s[nxt], srcs[nxt], dsts[nxt])
                    _compute_ex(srcs[nxt], dsts[nxt], exs[nxt],
                                als_v, ald_v, c_v)
                pltpu.make_async_copy(
                    h_hbm.at[idx4s[cur].at[0]], rowsa, semga).wait()
                _scale_rows_half(rowsa, exs[cur], 0)
                da = pltpu.async_copy(rowsa, acc.at[idx4s[cur].at[2]],
                                      semsca, add=True)
                pltpu.make_async_copy(
                    h_hbm.at[idx4s[cur].at[1]], rowsb, semgb).wait()
                _scale_rows_half(rowsb, exs[cur], 1)
                db = pltpu.async_copy(rowsb, acc.at[idx4s[cur].at[3]],
                                      semscb, add=True)
                dd = pltpu.async_copy(exs[cur], den.at[dsts[cur]], semd,
                                      add=True)
                da.wait()
                if has_next:
                    pltpu.async_copy(h_hbm.at[idx4s[nxt].at[0]], rowsa, semga)
                db.wait()
                if has_next:
                    pltpu.async_copy(h_hbm.at[idx4s[nxt].at[1]], rowsb, semgb)
                dd.wait()
                if next2_pred is not None:
                    @pl.when(next2_pred)
                    def _():
                        off = ebase + (g + 2) * K
                        pltpu.async_copy(src_hbm.at[pl.ds(off, K)],
                                         srcs[cur], semi[cur])
                        pltpu.async_copy(dst_hbm.at[pl.ds(off, K)],
                                         dsts[cur], semi[cur])

            pltpu.sync_copy(src_hbm.at[pl.ds(ebase, K)], src0)
            pltpu.sync_copy(dst_hbm.at[pl.ds(ebase, K)], dst0)
            _fill_idx4(idx4a, src0, dst0)
            pltpu.async_copy(h_hbm.at[idx4a.at[0]], rowsa, semga)
            pltpu.async_copy(h_hbm.at[idx4a.at[1]], rowsb, semgb)
            pltpu.async_copy(src_hbm.at[pl.ds(ebase + K, K)], src1, semi1)
            pltpu.async_copy(dst_hbm.at[pl.ds(ebase + K, K)], dst1, semi1)
            _compute_ex(src0, dst0, ex0, als_v, ald_v, c_v)

            def pair(tt, _):
                g0 = 2 * tt
                step(g0, 0, 1, True, g0 + 2 < ch)
                step(g0 + 1, 1, 0, True, g0 + 3 < ch)
                return 0
            lax.fori_loop(0, ch // 2, pair, 0)
            step(ch - 1, 0, 1, False, None)

        @pl.when(cid == 0)
        def _():
            run_pipeline(sid * (K * CH0), CH0)

        @pl.when(cid == 1)
        def _():
            run_pipeline(16 * K * CH0 + sid * (K * CH1), CH1)

        plsc.subcore_barrier()
        pltpu.sync_copy(acc.at[pl.ds(row0, ROWS_PT)],
                        p_out.at[cid, pl.ds(row0, ROWS_PT)])
        for t in range(NP // K // 16 + 1):
            cchunk = sid + 16 * t

            @pl.when(jnp.logical_and(cchunk < NP // K, cid == 0))
            def _():
                pltpu.sync_copy(den.at[pl.ds(cchunk * K, K)],
                                d0_out.at[pl.ds(cchunk * K, K)])

            @pl.when(jnp.logical_and(cchunk < NP // K, cid == 1))
            def _():
                pltpu.sync_copy(den.at[pl.ds(cchunk * K, K)],
                                d1_out.at[pl.ds(cchunk * K, K)])

    return k(h, als, ald, ctab, srcp, dstp)


def kernel(x, edge_index, batch, W1, a_src1, a_dst1, b1, W2, a_src2, a_dst2, b2):
    loop = jnp.arange(N, dtype=jnp.int32)
    pad = jnp.full((EPAD - EL,), N, jnp.int32)
    srcp = jnp.concatenate([edge_index[0].astype(jnp.int32), loop, pad])
    dstp = jnp.concatenate([edge_index[1].astype(jnp.int32), loop, pad])
    xp = jnp.concatenate([x, jnp.zeros((NP - N, x.shape[1]), jnp.float32)], axis=0)
    batp = jnp.concatenate(
        [batch.astype(jnp.int32), jnp.full((NP - N,), NG, jnp.int32)]
    ).reshape(1, NP)

    h1, als1, ald1, c1 = _tc_prep1(xp, W1, a_src1[:, None], a_dst1[:, None])
    p1, d1a, d1b = _sc_edge_pass(
        h1, als1.reshape(NP), ald1.reshape(NP), c1.reshape(NP), srcp, dstp)
    h2, als2, ald2, c2 = _tc_prep2(p1, d1a[:, None], d1b[:, None], b1,
                                   W2, a_src2[:, None], a_dst2[:, None])
    p2, d2a, d2b = _sc_edge_pass(
        h2, als2.reshape(NP), ald2.reshape(NP), c2.reshape(NP), srcp, dstp)
    return _tc_pool(p2, d2a[:, None], d2b[:, None], b2, batp)
